# baseline (speedup 1.0000x reference)
import functools
import numpy as np
import jax
import jax.numpy as jnp
from jax import lax
from jax.experimental import pallas as pl
from jax.experimental.pallas import tpu as pltpu
from jax.experimental.pallas import tpu_sc as plsc

_MODEL_DIM = 64
_MAX_POS = 512


def _positional_encoding(position, model_dim):
    pos = np.arange(position)[:, np.newaxis].astype(np.float32)
    i = np.arange(model_dim)[np.newaxis, :].astype(np.float32)
    angle_rates = 1.0 / np.power(10000, 2 * (i // 2) / np.float32(model_dim))
    angle_rads = pos * angle_rates
    angle_rads[:, 0::2] = np.sin(angle_rads[:, 0::2])
    angle_rads[:, 1::2] = np.cos(angle_rads[:, 1::2])
    return angle_rads.astype(np.float32)


_POS_ENC = _positional_encoding(_MAX_POS, _MODEL_DIM)


@functools.partial(jax.jit, static_argnums=(4, 5, 6))
def _embed(idx_pair, half_off, table2, pos, batch, seq, dim):
    B = batch * seq
    NC, NS = 2, 16
    NW = NC * NS
    rows_per_w = B // NW
    chunk = 128
    n_chunks = rows_per_w // chunk
    nvec = dim // 16
    scale = float(np.sqrt(dim))

    mesh = plsc.VectorSubcoreMesh(core_axis_name="c", subcore_axis_name="s")

    @functools.partial(
        pl.kernel,
        mesh=mesh,
        compiler_params=pltpu.CompilerParams(needs_layout_passes=False),
        out_type=jax.ShapeDtypeStruct((B, dim), jnp.float32),
        scratch_types=[
            pltpu.VMEM((chunk,), jnp.int32),
            pltpu.VMEM((chunk,), jnp.int32),
            pltpu.VMEM((chunk, 2 * dim), jnp.float32),
            pltpu.VMEM((chunk, dim), jnp.float32),
            pltpu.VMEM((seq, dim), jnp.float32),
            pltpu.SemaphoreType.DMA,
        ],
    )
    def k(ip_hbm, ho_hbm, table_hbm, pos_hbm, out_hbm,
          ip_c, ho_c, rows2_v, out_v, pos_v, sem):
        wid = lax.axis_index("s") * NC + lax.axis_index("c")
        base = wid * rows_per_w
        pltpu.sync_copy(pos_hbm, pos_v)

        def chunk_body(c, carry):
            off = c * chunk
            pltpu.sync_copy(ip_hbm.at[pl.ds(base + off, chunk)], ip_c)
            pltpu.sync_copy(ho_hbm.at[pl.ds(base + off, chunk)], ho_c)
            pltpu.async_copy(table_hbm.at[ip_c], rows2_v, sem).wait()

            start = lax.rem(off, seq)

            def grp_body(g, carry2):
                v = ho_c[pl.ds(g * 16, 16)]
                for l in range(16):
                    kk = g * 16 + l
                    ho = v[l]
                    prow = lax.rem(start + kk, seq)
                    for j in range(nvec):
                        sl = pl.ds(j * 16, 16)
                        out_v[kk, sl] = (
                            rows2_v[kk, pl.ds(ho + j * 16, 16)] * scale
                            + pos_v[prow, sl]
                        )
                return carry2

            lax.fori_loop(0, chunk // 16, grp_body, 0)
            pltpu.sync_copy(out_v, out_hbm.at[pl.ds(base + off, chunk)])
            return carry

        lax.fori_loop(0, n_chunks, chunk_body, 0)

    return k(idx_pair, half_off, table2, pos)


def kernel(inp, table, training):
    batch, seq = inp.shape
    vocab, dim = table.shape
    idx = inp.reshape(-1)
    idx_pair = lax.shift_right_logical(idx, 1)
    half_off = (idx & 1) * dim
    table2 = table.reshape(vocab // 2, 2 * dim)
    pos = jnp.asarray(_POS_ENC[:seq])
    out = _embed(idx_pair, half_off, table2, pos, batch, seq, dim)
    return out.reshape(batch, seq, dim)


# speedup vs baseline: 1.0847x; 1.0847x over previous
import functools
import numpy as np
import jax
import jax.numpy as jnp
from jax import lax
from jax.experimental import pallas as pl
from jax.experimental.pallas import tpu as pltpu
from jax.experimental.pallas import tpu_sc as plsc

_MODEL_DIM = 64
_MAX_POS = 512


def _positional_encoding(position, model_dim):
    pos = np.arange(position)[:, np.newaxis].astype(np.float32)
    i = np.arange(model_dim)[np.newaxis, :].astype(np.float32)
    angle_rates = 1.0 / np.power(10000, 2 * (i // 2) / np.float32(model_dim))
    angle_rads = pos * angle_rates
    angle_rads[:, 0::2] = np.sin(angle_rads[:, 0::2])
    angle_rads[:, 1::2] = np.cos(angle_rads[:, 1::2])
    return angle_rads.astype(np.float32)


_POS_ENC = _positional_encoding(_MAX_POS, _MODEL_DIM)


@functools.partial(jax.jit, static_argnums=(4, 5, 6))
def _embed(idx_pair, half_off, table2, pos, batch, seq, dim):
    B = batch * seq
    NC, NS = 2, 16
    NW = NC * NS
    rows_per_w = B // NW
    chunk = 256
    n_chunks = rows_per_w // chunk
    nvec = dim // 16
    scale = float(np.sqrt(dim))

    mesh = plsc.VectorSubcoreMesh(core_axis_name="c", subcore_axis_name="s")

    @functools.partial(
        pl.kernel,
        mesh=mesh,
        compiler_params=pltpu.CompilerParams(needs_layout_passes=False),
        out_type=jax.ShapeDtypeStruct((B, dim), jnp.float32),
        scratch_types=[
            pltpu.VMEM((chunk,), jnp.int32),
            pltpu.VMEM((chunk,), jnp.int32),
            pltpu.VMEM((chunk, 2 * dim), jnp.float32),
            pltpu.VMEM((chunk, dim), jnp.float32),
            pltpu.VMEM((seq, dim), jnp.float32),
            pltpu.SemaphoreType.DMA,
        ],
    )
    def k(ip_hbm, ho_hbm, table_hbm, pos_hbm, out_hbm,
          ip_c, ho_c, rows2_v, out_v, pos_v, sem):
        wid = lax.axis_index("s") * NC + lax.axis_index("c")
        base = wid * rows_per_w
        pltpu.sync_copy(pos_hbm, pos_v)

        def chunk_body(c, carry):
            off = c * chunk
            pltpu.sync_copy(ip_hbm.at[pl.ds(base + off, chunk)], ip_c)
            pltpu.sync_copy(ho_hbm.at[pl.ds(base + off, chunk)], ho_c)
            pltpu.async_copy(table_hbm.at[ip_c], rows2_v, sem).wait()

            start = lax.rem(off, seq)

            def grp_body(g, carry2):
                v = ho_c[pl.ds(g * 16, 16)]
                for l in range(16):
                    kk = g * 16 + l
                    ho = v[l]
                    prow = lax.rem(start + kk, seq)
                    for j in range(nvec):
                        sl = pl.ds(j * 16, 16)
                        out_v[kk, sl] = (
                            rows2_v[kk, pl.ds(ho + j * 16, 16)] * scale
                            + pos_v[prow, sl]
                        )
                return carry2

            lax.fori_loop(0, chunk // 16, grp_body, 0)
            pltpu.sync_copy(out_v, out_hbm.at[pl.ds(base + off, chunk)])
            return carry

        lax.fori_loop(0, n_chunks, chunk_body, 0)

    return k(idx_pair, half_off, table2, pos)


def kernel(inp, table, training):
    batch, seq = inp.shape
    vocab, dim = table.shape
    idx = inp.reshape(-1)
    idx_pair = lax.shift_right_logical(idx, 1)
    half_off = (idx & 1) * dim
    table2 = table.reshape(vocab // 2, 2 * dim)
    pos = jnp.asarray(_POS_ENC[:seq])
    out = _embed(idx_pair, half_off, table2, pos, batch, seq, dim)
    return out.reshape(batch, seq, dim)


# speedup vs baseline: 1.2200x; 1.1247x over previous
import functools
import numpy as np
import jax
import jax.numpy as jnp
from jax import lax
from jax.experimental import pallas as pl
from jax.experimental.pallas import tpu as pltpu
from jax.experimental.pallas import tpu_sc as plsc

_MODEL_DIM = 64
_MAX_POS = 512


def _positional_encoding(position, model_dim):
    pos = np.arange(position)[:, np.newaxis].astype(np.float32)
    i = np.arange(model_dim)[np.newaxis, :].astype(np.float32)
    angle_rates = 1.0 / np.power(10000, 2 * (i // 2) / np.float32(model_dim))
    angle_rads = pos * angle_rates
    angle_rads[:, 0::2] = np.sin(angle_rads[:, 0::2])
    angle_rads[:, 1::2] = np.cos(angle_rads[:, 1::2])
    return angle_rads.astype(np.float32)


_POS_ENC = _positional_encoding(_MAX_POS, _MODEL_DIM)


@functools.partial(jax.jit, static_argnums=(2,))
def _pair_gather(idx_pair, table2, B):
    NC, NS = 2, 16
    NW = NC * NS
    rows_per_w = B // NW
    chunk = 256
    n_chunks = rows_per_w // chunk
    dim2 = table2.shape[1]

    mesh = plsc.VectorSubcoreMesh(core_axis_name="c", subcore_axis_name="s")

    @functools.partial(
        pl.kernel,
        mesh=mesh,
        compiler_params=pltpu.CompilerParams(needs_layout_passes=False),
        out_type=jax.ShapeDtypeStruct((B, dim2), jnp.float32),
        scratch_types=[
            pltpu.VMEM((chunk,), jnp.int32),
            pltpu.VMEM((chunk, dim2), jnp.float32),
            pltpu.SemaphoreType.DMA,
        ],
    )
    def k(ip_hbm, table_hbm, out_hbm, ip_c, rows2_v, sem):
        wid = lax.axis_index("s") * NC + lax.axis_index("c")
        base = wid * rows_per_w

        def chunk_body(c, carry):
            off = c * chunk
            pltpu.sync_copy(ip_hbm.at[pl.ds(base + off, chunk)], ip_c)
            pltpu.async_copy(table_hbm.at[ip_c], rows2_v, sem).wait()
            pltpu.sync_copy(rows2_v, out_hbm.at[pl.ds(base + off, chunk)])
            return carry

        lax.fori_loop(0, n_chunks, chunk_body, 0)

    return k(idx_pair, table2)


def _select_kernel(g2_ref, idx_ref, pos_ref, out_ref, *, dim, scale):
    g2 = g2_ref[...]
    idx = idx_ref[...]
    pos = pos_ref[...]
    lo = g2[:, :, :dim]
    hi = g2[:, :, dim:]
    par = (idx & 1)[:, :, None] == 1
    out_ref[...] = jnp.where(par, hi, lo) * scale + pos


@functools.partial(jax.jit, static_argnums=(3, 4, 5))
def _select_scale_add(g2, idx, pos, batch, seq, dim):
    bb = 64
    scale = float(np.sqrt(dim))
    return pl.pallas_call(
        functools.partial(_select_kernel, dim=dim, scale=scale),
        grid=(batch // bb,),
        in_specs=[
            pl.BlockSpec((bb, seq, 2 * dim), lambda i: (i, 0, 0)),
            pl.BlockSpec((bb, seq), lambda i: (i, 0)),
            pl.BlockSpec((1, seq, dim), lambda i: (0, 0, 0)),
        ],
        out_specs=pl.BlockSpec((bb, seq, dim), lambda i: (i, 0, 0)),
        out_shape=jax.ShapeDtypeStruct((batch, seq, dim), jnp.float32),
    )(g2, idx, pos)


def kernel(inp, table, training):
    batch, seq = inp.shape
    vocab, dim = table.shape
    idx = inp.reshape(-1)
    idx_pair = lax.shift_right_logical(idx, 1)
    table2 = table.reshape(vocab // 2, 2 * dim)
    g2 = _pair_gather(idx_pair, table2, batch * seq)
    pos = jnp.asarray(_POS_ENC[:seq])[None]
    return _select_scale_add(
        g2.reshape(batch, seq, 2 * dim), inp, pos, batch, seq, dim
    )
